# Initial kernel scaffold; baseline (speedup 1.0000x reference)
import jax
import jax.numpy as jnp
from jax.experimental import pallas as pl


def kernel(edge_item_idx, edge_rel_idx, edge_ent_idx, user_emb, item_emb, entity_emb, relation_emb, intent_emb):
    raise NotImplementedError("write your pallas kernel here")


# speedup vs baseline: 1.2030x; 1.2030x over previous
import jax
import jax.numpy as jnp
from jax import lax
from jax.experimental import pallas as pl
from jax.experimental.pallas import tpu as pltpu
from jax.experimental.pallas import tpu_sc as plsc

E = 800_000
D = 64
CH = 1024
BLK = 128
NCHUNKS = 800
EPAD = NCHUNKS * CH
CHUNKS_PER_SUB = NCHUNKS // 16
QUARTER = 12_500
NPASS = 2
ACC_ROWS = 12_544
ROWS_PER_SUB = ACC_ROWS // 16
CNT_ROWS = 896
HIST = CNT_ROWS * 16
ALPHA = 0.6
BETA = 0.3


def _sc_body(ent_t, rel_t, iidx_h, ridx_h, eidx_h, zmsg_h, zcnt_h,
             msg_out, cnt_out,
             iidx_v, ridx_v, eidx_v, eidx_c, ridx_c, midx_c,
             ent_b, rel_b, hist, iota_v, acc, cacc, sem_g, sem_s):
    c = lax.axis_index("c")
    s = lax.axis_index("s")
    chunk0 = s * CHUNKS_PER_SUB
    i32 = jnp.int32
    lane = lax.iota(i32, 16)
    ones16 = jnp.full((16,), 1.0, jnp.float32)
    zero16 = jnp.zeros((16,), i32)
    dummy16 = jnp.full((16,), QUARTER, i32)

    for k in range(CNT_ROWS // 16):
        iota_v[pl.ds(k * 16, 16)] = lane + (k * 16)

    for p in range(NPASS):
        q = c * NPASS + p
        base = q * QUARTER

        pltpu.sync_copy(zmsg_h, acc.at[pl.ds(s * ROWS_PER_SUB, ROWS_PER_SUB)])
        pltpu.sync_copy(zcnt_h, hist)

        @pl.when(s == 0)
        def _():
            pltpu.sync_copy(zcnt_h, cacc)

        plsc.subcore_barrier()

        @pl.loop(0, CHUNKS_PER_SUB)
        def _(g):
            chunk = chunk0 + g
            pltpu.sync_copy(iidx_h.at[chunk], iidx_v)
            pltpu.sync_copy(ridx_h.at[chunk], ridx_v)
            pltpu.sync_copy(eidx_h.at[chunk], eidx_v)

            @pl.loop(0, CH // 16, init_carry=0)
            def compress(k, off):
                sl = pl.ds(k * 16, 16)
                loc = iidx_v[sl] - base
                valid = (loc >= 0) & (loc < QUARTER)
                cs = plsc.cumsum(jnp.where(valid, 1, 0))
                pos = jnp.where(valid, off + cs - 1, CH + BLK + lane)
                plsc.store_scatter(eidx_c, [pos], eidx_v[sl])
                plsc.store_scatter(ridx_c, [pos], ridx_v[sl])
                plsc.store_scatter(midx_c, [pos], loc)
                hidx = jnp.where(valid, loc, ACC_ROWS + lane)
                plsc.addupdate_scatter(
                    hist,
                    [lax.shift_right_logical(hidx, 4),
                     lax.bitwise_and(hidx, 15)],
                    ones16)
                return off + jnp.max(cs)

            off = compress
            for k in range(BLK // 16):
                sl = pl.ds(off + k * 16, 16)
                eidx_c[sl] = zero16
                ridx_c[sl] = zero16
                midx_c[sl] = dummy16

            nb = lax.shift_right_logical(off + (BLK - 1), 7)

            @pl.loop(0, nb)
            def blocks(b):
                o = b * BLK
                g1 = pltpu.async_copy(
                    ent_t.at[eidx_c.at[pl.ds(o, BLK)]], ent_b, sem_g)
                g2 = pltpu.async_copy(
                    rel_t.at[ridx_c.at[pl.ds(o, BLK)]], rel_b, sem_g)
                g1.wait()
                g2.wait()
                s1 = pltpu.async_copy(
                    ent_b, acc.at[midx_c.at[pl.ds(o, BLK)]], sem_s, add=True)
                s2 = pltpu.async_copy(
                    rel_b, acc.at[midx_c.at[pl.ds(o, BLK)]], sem_s, add=True)
                s1.wait()
                s2.wait()

        for b in range(CNT_ROWS // BLK):
            pltpu.async_copy(hist.at[pl.ds(b * BLK, BLK)],
                             cacc.at[iota_v.at[pl.ds(b * BLK, BLK)]],
                             sem_s, add=True).wait()

        plsc.subcore_barrier()
        pltpu.sync_copy(acc.at[pl.ds(s * ROWS_PER_SUB, ROWS_PER_SUB)],
                        msg_out.at[pl.ds(q * ACC_ROWS + s * ROWS_PER_SUB,
                                         ROWS_PER_SUB)])

        @pl.when(s == 0)
        def _():
            pltpu.sync_copy(cacc, cnt_out.at[pl.ds(q * CNT_ROWS, CNT_ROWS)])

        plsc.subcore_barrier()


def _sc_aggregate(ent_t, rel_t, iidx2, ridx2, eidx2, zmsg, zcnt):
    mesh = plsc.VectorSubcoreMesh(core_axis_name="c", subcore_axis_name="s")
    fn = pl.kernel(
        _sc_body,
        out_type=(
            jax.ShapeDtypeStruct((4 * ACC_ROWS, D), jnp.float32),
            jax.ShapeDtypeStruct((4 * CNT_ROWS, 16), jnp.float32),
        ),
        mesh=mesh,
        compiler_params=pltpu.CompilerParams(use_tc_tiling_on_sc=False, needs_layout_passes=False),
        scratch_types=[
            pltpu.VMEM((CH,), jnp.int32),
            pltpu.VMEM((CH,), jnp.int32),
            pltpu.VMEM((CH,), jnp.int32),
            pltpu.VMEM((CH + 2 * BLK,), jnp.int32),
            pltpu.VMEM((CH + 2 * BLK,), jnp.int32),
            pltpu.VMEM((CH + 2 * BLK,), jnp.int32),
            pltpu.VMEM((BLK, D), jnp.float32),
            pltpu.VMEM((BLK, D), jnp.float32),
            pltpu.VMEM((CNT_ROWS, 16), jnp.float32),
            pltpu.VMEM((CNT_ROWS,), jnp.int32),
            pltpu.VMEM_SHARED((ACC_ROWS, D), jnp.float32),
            pltpu.VMEM_SHARED((CNT_ROWS, 16), jnp.float32),
            pltpu.SemaphoreType.DMA,
            pltpu.SemaphoreType.DMA,
        ],
    )
    return fn(ent_t, rel_t, iidx2, ridx2, eidx2, zmsg, zcnt)


def _finish_body(item_ref, agg_ref, cnt_ref, intent_ref, out_ref):
    item = item_ref[...]
    intent = intent_ref[...]
    logits = jnp.dot(item, intent.T, preferred_element_type=jnp.float32)
    m = jnp.max(logits, axis=1, keepdims=True)
    e = jnp.exp(logits - m)
    att = e / jnp.sum(e, axis=1, keepdims=True)
    intent_item = jnp.dot(att, intent, preferred_element_type=jnp.float32)
    kg = agg_ref[...] / jnp.maximum(cnt_ref[...], 1.0)
    out_ref[...] = item + ALPHA * kg + BETA * intent_item


def _finish(item_emb, agg, cnt, intent_emb):
    n, d = item_emb.shape
    blk = 1000
    grid = (n // blk,)
    return pl.pallas_call(
        _finish_body,
        out_shape=jax.ShapeDtypeStruct((n, d), jnp.float32),
        grid=grid,
        in_specs=[
            pl.BlockSpec((blk, d), lambda i: (i, 0)),
            pl.BlockSpec((blk, d), lambda i: (i, 0)),
            pl.BlockSpec((blk, 1), lambda i: (i, 0)),
            pl.BlockSpec((4, d), lambda i: (0, 0)),
        ],
        out_specs=pl.BlockSpec((blk, d), lambda i: (i, 0)),
    )(item_emb, agg, cnt, intent_emb)


def kernel(edge_item_idx, edge_rel_idx, edge_ent_idx, user_emb, item_emb,
           entity_emb, relation_emb, intent_emb):
    f32 = jnp.float32
    i32 = jnp.int32
    pad = EPAD - E
    iidx2 = jnp.concatenate(
        [edge_item_idx.astype(i32), jnp.full((pad,), 2 ** 28, i32)]
    ).reshape(NCHUNKS, CH)
    ridx2 = jnp.concatenate(
        [edge_rel_idx.astype(i32), jnp.zeros((pad,), i32)]
    ).reshape(NCHUNKS, CH)
    eidx2 = jnp.concatenate(
        [edge_ent_idx.astype(i32), jnp.zeros((pad,), i32)]
    ).reshape(NCHUNKS, CH)
    zmsg = jnp.zeros((ROWS_PER_SUB, D), f32)
    zcnt = jnp.zeros((CNT_ROWS, 16), f32)

    msg, cntp = _sc_aggregate(entity_emb, relation_emb,
                              iidx2, ridx2, eidx2, zmsg, zcnt)

    agg = jnp.concatenate(
        [msg[q * ACC_ROWS:q * ACC_ROWS + QUARTER] for q in range(4)], axis=0)
    cflat = cntp.reshape(4, CNT_ROWS * 16)
    cnt = jnp.concatenate([cflat[q, :QUARTER] for q in range(4)])[:, None]

    item_out = _finish(item_emb, agg, cnt, intent_emb)
    return (user_emb, item_out)
